# Initial kernel scaffold; baseline (speedup 1.0000x reference)
import jax
import jax.numpy as jnp
from jax.experimental import pallas as pl


def kernel(feats, edge_index, time_table, W_self1, b_self1, W_neigh1, W_self2, b_self2, W_neigh2, W_struct, b_struct, W_r, b_r, W_z, b_z, W_c, b_c, W_fg, b_fg):
    raise NotImplementedError("write your pallas kernel here")


# speedup vs baseline: 4.2907x; 4.2907x over previous
import functools

import jax
import jax.numpy as jnp
from jax import lax
from jax.experimental import pallas as pl
from jax.experimental.pallas import tpu as pltpu
from jax.experimental.pallas import tpu_sc as plsc

T = 3
N = 10000
E = 320000
D = 128
TIME_DIM = 32
STRUCT = 16

NC = 2
NS = 16
NW = NC * NS
EPW = E // NW
CH = 80
NCH = EPW // CH
RPT = 640
RPT_LAST = N - RPT * (NS - 1)

BL = 1000


def _make_segsum(with_deg: bool):
  mesh = plsc.VectorSubcoreMesh(core_axis_name="c", subcore_axis_name="s",
                                num_cores=NC, num_subcores=NS)
  out_type = [jax.ShapeDtypeStruct((T * NC, N, D), jnp.float32)]
  if with_deg:
    out_type.append(jax.ShapeDtypeStruct((T * NC * N,), jnp.float32))
  scratch = [
      pltpu.VMEM((CH,), jnp.int32),
      pltpu.VMEM((CH,), jnp.int32),
      pltpu.VMEM((CH, D), jnp.float32),
      pltpu.VMEM_SHARED((N, D), jnp.float32),
      pltpu.SemaphoreType.DMA,
  ]
  if with_deg:
    scratch += [
        pltpu.VMEM((CH,), jnp.float32),
        pltpu.VMEM((RPT,), jnp.float32),
        pltpu.VMEM_SHARED((N,), jnp.float32),
    ]

  def body(table, srcf, dstf, znd, zf, onesf, *refs):
    if with_deg:
      out_agg, out_deg, sidx, didx, rows, acc, sem, ones1, dstg, dacc = refs
    else:
      out_agg, sidx, didx, rows, acc, sem = refs
    c = lax.axis_index("c")
    s = lax.axis_index("s")
    wid = s * NC + c

    def each_span(fn):
      @pl.when(s < NS - 1)
      def _():
        fn(s * RPT, RPT)

      @pl.when(s == NS - 1)
      def _():
        fn((NS - 1) * RPT, RPT_LAST)

    if with_deg:
      pltpu.sync_copy(onesf, ones1)

    for t in range(T):
      pltpu.sync_copy(znd, rows)

      def zero_span(off, sz):
        for j in range(sz // CH):
          pltpu.sync_copy(rows, acc.at[pl.ds(off + j * CH, CH)])
        if with_deg:
          pltpu.sync_copy(zf.at[pl.ds(0, sz)], dstg.at[pl.ds(0, sz)])
          pltpu.sync_copy(dstg.at[pl.ds(0, sz)], dacc.at[pl.ds(off, sz)])

      each_span(zero_span)
      plsc.subcore_barrier()

      def step(i, carry):
        base = t * E + wid * EPW + i * CH
        pltpu.sync_copy(srcf.at[pl.ds(base, CH)], sidx)
        pltpu.sync_copy(dstf.at[pl.ds(base, CH)], didx)
        pltpu.async_copy(table.at[sidx], rows, sem).wait()
        pltpu.sync_copy(rows, acc.at[didx], add=True)
        if with_deg:
          pltpu.sync_copy(ones1, dacc.at[didx], add=True)
        return carry

      lax.fori_loop(0, NCH, step, 0)
      plsc.subcore_barrier()

      q = t * NC + c

      def write_span(off, sz):
        for j in range(sz // CH):
          o = off + j * CH
          pltpu.sync_copy(acc.at[pl.ds(o, CH)], rows)
          pltpu.sync_copy(rows, out_agg.at[q, pl.ds(o, CH)])
        if with_deg:
          pltpu.sync_copy(dacc.at[pl.ds(off, sz)], dstg.at[pl.ds(0, sz)])
          pltpu.sync_copy(dstg.at[pl.ds(0, sz)],
                          out_deg.at[pl.ds(q * N + off, sz)])

      each_span(write_span)
      plsc.subcore_barrier()

  return pl.kernel(body, out_type=tuple(out_type), mesh=mesh,
                   scratch_types=scratch)


@functools.cache
def _segsum_deg():
  return _make_segsum(True)


@functools.cache
def _segsum():
  return _make_segsum(False)


def _dot(a, b):
  return jnp.dot(a, b, preferred_element_type=jnp.float32)


def _layer1_body(f_ref, tr_ref, ws_ref, bs_ref, wn_ref, hs_ref, hn_ref):
  f = f_ref[0]
  tr = tr_ref[pl.ds(pl.program_id(0), 1), :]
  hs = _dot(f, ws_ref[:D, :]) + _dot(tr, ws_ref[D:, :]) + bs_ref[...]
  hn = _dot(f, wn_ref[:D, :]) + _dot(tr, wn_ref[D:, :])
  hs_ref[0] = hs
  hn_ref[0] = hn


def _layer2_body(hs1_ref, agg_ref, dinv_ref, ws_ref, bs_ref, wn_ref,
                 hs2_ref, hn2_ref):
  a = agg_ref[0] + agg_ref[1]
  h1 = jnp.maximum(hs1_ref[0] + a * dinv_ref[0, :, :1], 0.0)
  hs2_ref[0] = _dot(h1, ws_ref[...]) + bs_ref[...]
  hn2_ref[0] = _dot(h1, wn_ref[...])


def _gru_body(hs2_ref, agg_ref, dinv_ref, degb_ref, wst_ref, bst_ref,
              wr_ref, br_ref, wz_ref, bz_ref, wc_ref, bc_ref,
              wfg_ref, bfg_ref, out_ref):
  h = jnp.zeros((BL, D), jnp.float32)
  for t in range(T):
    a = agg_ref[NC * t] + agg_ref[NC * t + 1]
    x = hs2_ref[t] + a * dinv_ref[t, :, :1]
    d = degb_ref[t, :, :1]
    struct = jnp.maximum(d * wst_ref[0] + bst_ref[...], 0.0)
    r = jax.nn.sigmoid(_dot(x, wr_ref[:D, :]) + _dot(h, wr_ref[D:2 * D, :])
                       + _dot(struct, wr_ref[2 * D:, :]) + br_ref[...])
    z = jax.nn.sigmoid(_dot(x, wz_ref[:D, :]) + _dot(h, wz_ref[D:2 * D, :])
                       + _dot(struct, wz_ref[2 * D:, :]) + bz_ref[...])
    cc = jnp.tanh(_dot(x, wc_ref[:D, :]) + _dot(r * h, wc_ref[D:2 * D, :])
                  + _dot(struct, wc_ref[2 * D:, :]) + bc_ref[...])
    fg = jax.nn.sigmoid(_dot(x, wfg_ref[:D, :])
                        + _dot(struct, wfg_ref[D:, :]) + bfg_ref[...])
    h = fg * ((1.0 - z) * h + z * cc)
    out_ref[t] = h


def _full(shape):
  return pl.BlockSpec(shape, lambda *_: tuple(0 for _ in shape))


def kernel(feats, edge_index, time_table, W_self1, b_self1, W_neigh1,
           W_self2, b_self2, W_neigh2, W_struct, b_struct, W_r, b_r,
           W_z, b_z, W_c, b_c, W_fg, b_fg):
  f32 = jnp.float32
  src = edge_index[:, 0, :].astype(jnp.int32)
  dst = edge_index[:, 1, :].astype(jnp.int32)
  srcf = (src + (jnp.arange(T, dtype=jnp.int32) * N)[:, None]).reshape(T * E)
  dstf = dst.reshape(T * E)
  tvecs = time_table[1:T + 1].astype(f32)
  znd = jnp.zeros((CH, D), f32)
  zf = jnp.zeros((RPT,), f32)
  onesf = jnp.ones((CH,), f32)

  grid2 = (T, N // BL)
  bspec = pl.BlockSpec((1, BL, D), lambda t, i: (t, i, 0))
  hs1, hn1 = pl.pallas_call(
      _layer1_body,
      grid=grid2,
      in_specs=[
          bspec,
          _full((T, TIME_DIM)),
          _full((D + TIME_DIM, D)),
          _full((1, D)),
          _full((D + TIME_DIM, D)),
      ],
      out_specs=[bspec, bspec],
      out_shape=[jax.ShapeDtypeStruct((T, N, D), f32)] * 2,
  )(feats, tvecs, W_self1, b_self1.reshape(1, D), W_neigh1)

  agg1, degs = _segsum_deg()(hn1.reshape(T * N, D), srcf, dstf, znd, zf,
                             onesf)
  degb = jnp.broadcast_to(
      degs.reshape(T, NC, N).sum(axis=1)[:, :, None], (T, N, 8))
  dinv = 1.0 / jnp.clip(degb, 1.0, None)

  aspec = pl.BlockSpec((NC, BL, D), lambda t, i: (t, i, 0))
  dspec = pl.BlockSpec((1, BL, 8), lambda t, i: (t, i, 0))
  hs2, hn2 = pl.pallas_call(
      _layer2_body,
      grid=grid2,
      in_specs=[bspec, aspec, dspec, _full((D, D)), _full((1, D)),
                _full((D, D))],
      out_specs=[bspec, bspec],
      out_shape=[jax.ShapeDtypeStruct((T, N, D), f32)] * 2,
  )(hs1, agg1, dinv, W_self2, b_self2.reshape(1, D), W_neigh2)

  (agg2,) = _segsum()(hn2.reshape(T * N, D), srcf, dstf, znd, zf, onesf)

  g_in = 2 * D + STRUCT
  outs = pl.pallas_call(
      _gru_body,
      grid=(N // BL,),
      in_specs=[
          pl.BlockSpec((T, BL, D), lambda i: (0, i, 0)),
          pl.BlockSpec((T * NC, BL, D), lambda i: (0, i, 0)),
          pl.BlockSpec((T, BL, 8), lambda i: (0, i, 0)),
          pl.BlockSpec((T, BL, 8), lambda i: (0, i, 0)),
          _full((1, STRUCT)), _full((1, STRUCT)),
          _full((g_in, D)), _full((1, D)),
          _full((g_in, D)), _full((1, D)),
          _full((g_in, D)), _full((1, D)),
          _full((D + STRUCT, D)), _full((1, D)),
      ],
      out_specs=pl.BlockSpec((T, BL, D), lambda i: (0, i, 0)),
      out_shape=jax.ShapeDtypeStruct((T, N, D), f32),
  )(hs2, agg2, dinv, degb, W_struct, b_struct.reshape(1, STRUCT),
    W_r, b_r.reshape(1, D), W_z, b_z.reshape(1, D), W_c, b_c.reshape(1, D),
    W_fg, b_fg.reshape(1, D))
  return outs
